# baseline (speedup 1.0000x reference)
import jax
import jax.numpy as jnp
from jax.experimental import pallas as pl


def _relu_kernel(x_ref, o_ref):
    o_ref[...] = jnp.maximum(x_ref[...], 0.0)


def kernel(x, edge_index, edge_attr, batch, W0, b0, Wm, We, bconv, Wroot, Wi, Wh, bl, W1, b1, W2, b2):
    N = x.shape[0]
    B = Wi.shape[0] // 2 * 0 + 512
    DIM = Wm.shape[0]
    h0 = x @ W0 + b0
    h = pl.pallas_call(
        _relu_kernel,
        out_shape=jax.ShapeDtypeStruct(h0.shape, h0.dtype),
        grid=(25,),
        in_specs=[pl.BlockSpec((2000, DIM), lambda i: (i, 0))],
        out_specs=pl.BlockSpec((2000, DIM), lambda i: (i, 0)),
    )(h0)
    src = edge_index[0]
    dst = edge_index[1]
    msg = jax.nn.relu(jnp.take(h, src, axis=0) @ Wm + edge_attr @ We + bconv)
    aggr = jax.ops.segment_sum(msg, dst, num_segments=N)
    h = jax.nn.relu(h @ Wroot + aggr)
    q_star = jnp.zeros((B, 2 * DIM), dtype=h.dtype)
    hs = jnp.zeros((B, DIM), dtype=h.dtype)
    cs = jnp.zeros((B, DIM), dtype=h.dtype)
    for _ in range(3):
        z = q_star @ Wi + hs @ Wh + bl
        i_g, f_g, g_g, o_g = jnp.split(z, 4, axis=-1)
        i_g = jax.nn.sigmoid(i_g)
        f_g = jax.nn.sigmoid(f_g)
        g_g = jnp.tanh(g_g)
        o_g = jax.nn.sigmoid(o_g)
        cs = f_g * cs + i_g * g_g
        hs = o_g * jnp.tanh(cs)
        q = hs
        e = jnp.sum(h * jnp.take(q, batch, axis=0), axis=-1)
        emax = jax.ops.segment_max(e, batch, num_segments=B)
        a = jnp.exp(e - jnp.take(emax, batch, axis=0))
        asum = jax.ops.segment_sum(a, batch, num_segments=B)
        a = a / (jnp.take(asum, batch, axis=0) + 1e-16)
        r = jax.ops.segment_sum(a[:, None] * h, batch, num_segments=B)
        q_star = jnp.concatenate([q, r], axis=-1)
    out = jax.nn.relu(q_star @ W1 + b1)
    out = out @ W2 + b2
    return out.reshape(-1)


# speedup vs baseline: 1.3685x; 1.3685x over previous
import functools

import jax
import jax.numpy as jnp
from jax import lax
from jax.experimental import pallas as pl
from jax.experimental.pallas import tpu as pltpu

N = 50000
E = 800000
DIN = 25
DIM = 64
FAB = 16
B = 512

RB = 2048
NBLK = 25
NPAD = RB * NBLK
EB = 8192
EBLK = 98
EPAD = EB * EBLK


def _pre_kernel(x_ref, W0_ref, b0_ref, Wm_ref, Wroot_ref, hm_ref, hr_ref):
    h0 = jnp.maximum(
        jnp.dot(x_ref[...], W0_ref[...], preferred_element_type=jnp.float32)
        + b0_ref[...], 0.0)
    hm_ref[...] = jnp.dot(h0, Wm_ref[...], preferred_element_type=jnp.float32)
    hr_ref[...] = jnp.dot(h0, Wroot_ref[...], preferred_element_type=jnp.float32)


def _pre(x_pad, W0, b0, Wm, Wroot):
    return pl.pallas_call(
        _pre_kernel,
        grid=(NBLK,),
        in_specs=[
            pl.BlockSpec((RB, DIN), lambda i: (i, 0)),
            pl.BlockSpec((DIN, DIM), lambda i: (0, 0)),
            pl.BlockSpec((1, DIM), lambda i: (0, 0)),
            pl.BlockSpec((DIM, DIM), lambda i: (0, 0)),
            pl.BlockSpec((DIM, DIM), lambda i: (0, 0)),
        ],
        out_specs=[
            pl.BlockSpec((RB, DIM), lambda i: (i, 0)),
            pl.BlockSpec((RB, DIM), lambda i: (i, 0)),
        ],
        out_shape=[
            jax.ShapeDtypeStruct((NPAD, DIM), jnp.float32),
            jax.ShapeDtypeStruct((NPAD, DIM), jnp.float32),
        ],
    )(x_pad, W0, b0.reshape(1, DIM), Wm, Wroot)


def _edge_t_kernel(ea_ref, We_ref, bc_ref, t_ref):
    t_ref[...] = (
        jnp.dot(ea_ref[...], We_ref[...], preferred_element_type=jnp.float32)
        + bc_ref[...])


def _edge_t(ea_pad, We, bconv):
    return pl.pallas_call(
        _edge_t_kernel,
        grid=(EBLK,),
        in_specs=[
            pl.BlockSpec((EB, FAB), lambda i: (i, 0)),
            pl.BlockSpec((FAB, DIM), lambda i: (0, 0)),
            pl.BlockSpec((1, DIM), lambda i: (0, 0)),
        ],
        out_specs=pl.BlockSpec((EB, DIM), lambda i: (i, 0)),
        out_shape=jax.ShapeDtypeStruct((EPAD, DIM), jnp.float32),
    )(ea_pad, We, bconv.reshape(1, DIM))


def _set2set_kernel(hr_ref, aggr_ref, batch_ref, Wi_ref, Wh_ref, bl_ref,
                    W1_ref, b1_ref, W2_ref, b2_ref, out_ref):
    Wi = Wi_ref[...]
    Wh = Wh_ref[...]
    bl = bl_ref[...]
    colid = lax.broadcasted_iota(jnp.int32, (B, RB), 0)

    def hblk(i):
        return jnp.maximum(
            hr_ref[pl.ds(i * RB, RB), :] + aggr_ref[pl.ds(i * RB, RB), :], 0.0)

    q_star = jnp.zeros((B, 2 * DIM), dtype=jnp.float32)
    hs = jnp.zeros((B, DIM), dtype=jnp.float32)
    cs = jnp.zeros((B, DIM), dtype=jnp.float32)
    for _ in range(3):
        z = (jnp.dot(q_star, Wi, preferred_element_type=jnp.float32)
             + jnp.dot(hs, Wh, preferred_element_type=jnp.float32) + bl)
        i_g = jax.nn.sigmoid(z[:, :DIM])
        f_g = jax.nn.sigmoid(z[:, DIM:2 * DIM])
        g_g = jnp.tanh(z[:, 2 * DIM:3 * DIM])
        o_g = jax.nn.sigmoid(z[:, 3 * DIM:])
        cs = f_g * cs + i_g * g_g
        hs = o_g * jnp.tanh(cs)
        q = hs

        def p1(i, emax):
            hb = hblk(i)
            S = lax.dot_general(q, hb, (((1,), (1,)), ((), ())),
                                preferred_element_type=jnp.float32)
            mask = batch_ref[pl.ds(i, 1), :] == colid
            Sm = jnp.where(mask, S, -jnp.inf)
            return jnp.maximum(emax, jnp.max(Sm, axis=1, keepdims=True))

        emax = lax.fori_loop(0, NBLK, p1, jnp.full((B, 1), -jnp.inf, jnp.float32))

        def p2(i, carry):
            asum, racc = carry
            hb = hblk(i)
            S = lax.dot_general(q, hb, (((1,), (1,)), ((), ())),
                                preferred_element_type=jnp.float32)
            mask = batch_ref[pl.ds(i, 1), :] == colid
            A = jnp.where(mask, jnp.exp(S - emax), 0.0)
            asum = asum + jnp.sum(A, axis=1, keepdims=True)
            racc = racc + jnp.dot(A, hb, preferred_element_type=jnp.float32)
            return asum, racc

        asum, racc = lax.fori_loop(
            0, NBLK, p2,
            (jnp.zeros((B, 1), jnp.float32), jnp.zeros((B, DIM), jnp.float32)))
        r = racc / (asum + 1e-16)
        q_star = jnp.concatenate([q, r], axis=1)

    o = jnp.maximum(
        jnp.dot(q_star, W1_ref[...], preferred_element_type=jnp.float32)
        + b1_ref[...], 0.0)
    out_ref[...] = (jnp.dot(o, W2_ref[...], preferred_element_type=jnp.float32)
                    + b2_ref[...])


def _set2set(hr_pad, aggr_pad, batch2d, Wi, Wh, bl, W1, b1, W2, b2):
    return pl.pallas_call(
        _set2set_kernel,
        out_shape=jax.ShapeDtypeStruct((B, 1), jnp.float32),
    )(hr_pad, aggr_pad, batch2d, Wi, Wh, bl.reshape(1, 4 * DIM),
      W1, b1.reshape(1, DIM), W2, b2.reshape(1, 1))


def kernel(x, edge_index, edge_attr, batch, W0, b0, Wm, We, bconv, Wroot,
           Wi, Wh, bl, W1, b1, W2, b2):
    x_pad = jnp.pad(x, ((0, NPAD - N), (0, 0)))
    ea_pad = jnp.pad(edge_attr, ((0, EPAD - E), (0, 0)))
    batch2d = jnp.pad(batch.astype(jnp.int32), (0, NPAD - N),
                      constant_values=B).reshape(NBLK, RB)

    hm, hr = _pre(x_pad, W0, b0, Wm, Wroot)
    t = _edge_t(ea_pad, We, bconv)

    src = edge_index[0]
    dst = edge_index[1]
    msg = jnp.maximum(jnp.take(hm[:N], src, axis=0) + t[:E], 0.0)
    aggr = jax.ops.segment_sum(msg, dst, num_segments=N)
    aggr_pad = jnp.pad(aggr, ((0, NPAD - N), (0, 0)))

    out = _set2set(hr, aggr_pad, batch2d, Wi, Wh, bl, W1, b1, W2, b2)
    return out.reshape(-1)


# speedup vs baseline: 2.9043x; 2.1223x over previous
import functools

import jax
import jax.numpy as jnp
from jax import lax
from jax.experimental import pallas as pl
from jax.experimental.pallas import tpu as pltpu
from jax.experimental.pallas import tpu_sc as plsc

N = 50000
E = 800000
DIN = 25
DIM = 64
FAB = 16
B = 512

NPAD = 50176
RB = 1792
NBLK = 28
NC = 2
NS = 16
NW = NC * NS
NR = NPAD // NC
TROWS = NR // NS

EB = 8192
EBLK = 98
EPAD = EB * EBLK
EPT = EPAD // NS
CH = 224
K = 2
SL = CH // K
NOUT = EPT // CH


def _pre_kernel(x_ref, W0_ref, b0_ref, Wm_ref, Wroot_ref, hm_ref, hr_ref):
    h0 = jnp.maximum(
        jnp.dot(x_ref[...], W0_ref[...], preferred_element_type=jnp.float32)
        + b0_ref[...], 0.0)
    hm_ref[...] = jnp.dot(h0, Wm_ref[...], preferred_element_type=jnp.float32)
    hr_ref[...] = jnp.dot(h0, Wroot_ref[...], preferred_element_type=jnp.float32)


def _pre(x_pad, W0, b0, Wm, Wroot):
    return pl.pallas_call(
        _pre_kernel,
        grid=(NBLK,),
        in_specs=[
            pl.BlockSpec((RB, DIN), lambda i: (i, 0)),
            pl.BlockSpec((DIN, DIM), lambda i: (0, 0)),
            pl.BlockSpec((1, DIM), lambda i: (0, 0)),
            pl.BlockSpec((DIM, DIM), lambda i: (0, 0)),
            pl.BlockSpec((DIM, DIM), lambda i: (0, 0)),
        ],
        out_specs=[
            pl.BlockSpec((RB, DIM), lambda i: (i, 0)),
            pl.BlockSpec((RB, DIM), lambda i: (i, 0)),
        ],
        out_shape=[
            jax.ShapeDtypeStruct((NPAD, DIM), jnp.float32),
            jax.ShapeDtypeStruct((NPAD, DIM), jnp.float32),
        ],
    )(x_pad, W0, b0.reshape(1, DIM), Wm, Wroot)


def _edge_t_kernel(ea_ref, We_ref, bc_ref, t_ref):
    t_ref[...] = (
        jnp.dot(ea_ref[...], We_ref[...], preferred_element_type=jnp.float32)
        + bc_ref[...])


def _edge_t(ea_pad, We, bconv):
    return pl.pallas_call(
        _edge_t_kernel,
        grid=(EBLK,),
        in_specs=[
            pl.BlockSpec((EB, FAB), lambda i: (i, 0)),
            pl.BlockSpec((FAB, DIM), lambda i: (0, 0)),
            pl.BlockSpec((1, DIM), lambda i: (0, 0)),
        ],
        out_specs=pl.BlockSpec((EB, DIM), lambda i: (i, 0)),
        out_shape=jax.ShapeDtypeStruct((EPAD, DIM), jnp.float32),
    )(ea_pad, We, bconv.reshape(1, DIM))


def _edge_sc_body(hm_hbm, srcp_hbm, dstp_hbm, t_hbm, out_hbm,
                  sbufs, dbufs, rbufs, tbufs, acc, gsem, tsem):
    c = lax.axis_index("c")
    s = lax.axis_index("s")
    w = c * NS + s

    def zrow(i, _):
        for k in range(K):
            for j in range(DIM // 16):
                rbufs[k][i, pl.ds(j * 16, 16)] = jnp.zeros((16,), jnp.float32)
        return 0
    lax.fori_loop(0, SL, zrow, 0)
    zoff = s * TROWS
    for q in range(TROWS // SL):
        pltpu.sync_copy(rbufs[q % K], acc.at[pl.ds(zoff + q * SL, SL)])

    @pl.when(s == 0)
    def _zero_dump():
        pltpu.sync_copy(rbufs[0].at[pl.ds(0, 8)], acc.at[pl.ds(NR, 8)])

    plsc.subcore_barrier()

    lo = c * NR
    ebase = s * EPT

    def outer(g, _):
        base = ebase + g * CH
        for k in range(K):
            pltpu.sync_copy(srcp_hbm.at[pl.ds(base + k * SL, SL)], sbufs[k])
            pltpu.sync_copy(dstp_hbm.at[pl.ds(base + k * SL, SL)], dbufs[k])
        gcps = [pltpu.async_copy(hm_hbm.at[sbufs[k]], rbufs[k], gsem)
                for k in range(K)]
        tcps = [pltpu.async_copy(t_hbm.at[pl.ds(base + k * SL, SL)],
                                 tbufs[k], tsem)
                for k in range(K)]

        for k in range(K):
            def dxf(i, _, _k=k):
                d = dbufs[_k][pl.ds(i * 16, 16)]
                loc = d - lo
                ok = (loc >= 0) & (loc < NR)
                dbufs[_k][pl.ds(i * 16, 16)] = jnp.where(ok, loc, NR)
                return 0
            lax.fori_loop(0, SL // 16, dxf, 0)

        for cp in gcps:
            cp.wait()
        for cp in tcps:
            cp.wait()

        for k in range(K):
            def cmp(i, _, _k=k):
                for j in range(DIM // 16):
                    v = (rbufs[_k][i, pl.ds(j * 16, 16)]
                         + tbufs[_k][i, pl.ds(j * 16, 16)])
                    rbufs[_k][i, pl.ds(j * 16, 16)] = jnp.maximum(v, 0.0)
                return 0
            lax.fori_loop(0, SL, cmp, 0)

        for k in range(K):
            pltpu.sync_copy(rbufs[k], acc.at[dbufs[k]], add=True)
        return 0

    lax.fori_loop(0, NOUT, outer, 0)

    plsc.subcore_barrier()

    pltpu.sync_copy(acc.at[pl.ds(s * TROWS, TROWS)],
                    out_hbm.at[pl.ds(c * NR + s * TROWS, TROWS)])


def _edge_sc(hm, srcp, dstp, t):
    mesh = plsc.VectorSubcoreMesh(core_axis_name="c", subcore_axis_name="s")
    f = functools.partial(
        pl.kernel,
        out_type=jax.ShapeDtypeStruct((NPAD, DIM), jnp.float32),
        mesh=mesh,
        compiler_params=pltpu.CompilerParams(use_tc_tiling_on_sc=False),
        scratch_types=[
            [pltpu.VMEM((SL,), jnp.int32) for _ in range(K)],
            [pltpu.VMEM((SL,), jnp.int32) for _ in range(K)],
            [pltpu.VMEM((SL, DIM), jnp.float32) for _ in range(K)],
            [pltpu.VMEM((SL, DIM), jnp.float32) for _ in range(K)],
            pltpu.VMEM_SHARED((NR + 8, DIM), jnp.float32),
            pltpu.SemaphoreType.DMA,
            pltpu.SemaphoreType.DMA,
        ],
    )(_edge_sc_body)
    return f(hm, srcp, dstp, t)


def _set2set_kernel(hr_ref, aggr_ref, batch_ref, Wi_ref, Wh_ref, bl_ref,
                    W1_ref, b1_ref, W2_ref, b2_ref, out_ref):
    Wi = Wi_ref[...]
    Wh = Wh_ref[...]
    bl = bl_ref[...]
    colid = lax.broadcasted_iota(jnp.int32, (B, RB), 0)

    def hblk(i):
        return jnp.maximum(
            hr_ref[pl.ds(i * RB, RB), :] + aggr_ref[pl.ds(i * RB, RB), :], 0.0)

    q_star = jnp.zeros((B, 2 * DIM), dtype=jnp.float32)
    hs = jnp.zeros((B, DIM), dtype=jnp.float32)
    cs = jnp.zeros((B, DIM), dtype=jnp.float32)
    for _ in range(3):
        z = (jnp.dot(q_star, Wi, preferred_element_type=jnp.float32)
             + jnp.dot(hs, Wh, preferred_element_type=jnp.float32) + bl)
        i_g = jax.nn.sigmoid(z[:, :DIM])
        f_g = jax.nn.sigmoid(z[:, DIM:2 * DIM])
        g_g = jnp.tanh(z[:, 2 * DIM:3 * DIM])
        o_g = jax.nn.sigmoid(z[:, 3 * DIM:])
        cs = f_g * cs + i_g * g_g
        hs = o_g * jnp.tanh(cs)
        q = hs

        def p1(i, emax):
            hb = hblk(i)
            S = lax.dot_general(q, hb, (((1,), (1,)), ((), ())),
                                preferred_element_type=jnp.float32)
            mask = batch_ref[pl.ds(i, 1), :] == colid
            Sm = jnp.where(mask, S, -jnp.inf)
            return jnp.maximum(emax, jnp.max(Sm, axis=1, keepdims=True))

        emax = lax.fori_loop(0, NBLK, p1, jnp.full((B, 1), -jnp.inf, jnp.float32))

        def p2(i, carry):
            asum, racc = carry
            hb = hblk(i)
            S = lax.dot_general(q, hb, (((1,), (1,)), ((), ())),
                                preferred_element_type=jnp.float32)
            mask = batch_ref[pl.ds(i, 1), :] == colid
            A = jnp.where(mask, jnp.exp(S - emax), 0.0)
            asum = asum + jnp.sum(A, axis=1, keepdims=True)
            racc = racc + jnp.dot(A, hb, preferred_element_type=jnp.float32)
            return asum, racc

        asum, racc = lax.fori_loop(
            0, NBLK, p2,
            (jnp.zeros((B, 1), jnp.float32), jnp.zeros((B, DIM), jnp.float32)))
        r = racc / (asum + 1e-16)
        q_star = jnp.concatenate([q, r], axis=1)

    o = jnp.maximum(
        jnp.dot(q_star, W1_ref[...], preferred_element_type=jnp.float32)
        + b1_ref[...], 0.0)
    out_ref[...] = (jnp.dot(o, W2_ref[...], preferred_element_type=jnp.float32)
                    + b2_ref[...])


def _set2set(hr_pad, aggr_pad, batch2d, Wi, Wh, bl, W1, b1, W2, b2):
    return pl.pallas_call(
        _set2set_kernel,
        out_shape=jax.ShapeDtypeStruct((B, 1), jnp.float32),
    )(hr_pad, aggr_pad, batch2d, Wi, Wh, bl.reshape(1, 4 * DIM),
      W1, b1.reshape(1, DIM), W2, b2.reshape(1, 1))


def kernel(x, edge_index, edge_attr, batch, W0, b0, Wm, We, bconv, Wroot,
           Wi, Wh, bl, W1, b1, W2, b2):
    x_pad = jnp.pad(x, ((0, NPAD - N), (0, 0)))
    ea_pad = jnp.pad(edge_attr, ((0, EPAD - E), (0, 0)))
    batch2d = jnp.pad(batch.astype(jnp.int32), (0, NPAD - N),
                      constant_values=B).reshape(NBLK, RB)
    srcp = jnp.pad(edge_index[0].astype(jnp.int32), (0, EPAD - E))
    dstp = jnp.pad(edge_index[1].astype(jnp.int32), (0, EPAD - E),
                   constant_values=1 << 30)

    hm, hr = _pre(x_pad, W0, b0, Wm, Wroot)
    t = _edge_t(ea_pad, We, bconv)
    aggr_pad = _edge_sc(hm, srcp, dstp, t)
    out = _set2set(hr, aggr_pad, batch2d, Wi, Wh, bl, W1, b1, W2, b2)
    return out.reshape(-1)
